# baseline (speedup 1.0000x reference)
import functools

import jax
import jax.numpy as jnp
from jax import lax
from jax.experimental import pallas as pl
from jax.experimental.pallas import tpu as pltpu
from jax.experimental.pallas import tpu_sc as plsc

NC = 2
NS = 16
NW = NC * NS

CHUNK = 64
NBUF = 2


def _proj_body(table_ref, w_ref, b_ref, out_ref):
    out_ref[...] = lax.dot_general(
        table_ref[...], w_ref[...],
        dimension_numbers=(((1,), (1,)), ((), ())),
        preferred_element_type=jnp.float32,
    ) + b_ref[...]


def _project_table(table, W, b):
    V, _ = table.shape
    D = W.shape[0]
    return pl.pallas_call(
        _proj_body,
        out_shape=jax.ShapeDtypeStruct((V, D), jnp.float32),
    )(table, W, b[None, :])


def _sc_gather(ptab, idx, n_total, d_model):
    b_per_w = n_total // NW
    n_chunks = b_per_w // CHUNK
    mesh = plsc.VectorSubcoreMesh(core_axis_name="c", subcore_axis_name="s")

    @functools.partial(
        pl.kernel,
        out_type=jax.ShapeDtypeStruct((n_total, d_model), jnp.float32),
        mesh=mesh,
        scratch_types=[
            pltpu.VMEM((n_chunks, CHUNK), jnp.int32),
            pltpu.VMEM((CHUNK, d_model), jnp.float32),
            pltpu.VMEM((CHUNK, d_model), jnp.float32),
            pltpu.SemaphoreType.DMA,
            pltpu.SemaphoreType.DMA,
            pltpu.SemaphoreType.DMA,
            pltpu.SemaphoreType.DMA,
        ],
    )
    def gather_kernel(ptab_hbm, idx_hbm, out_hbm, idx_v, rows0, rows1,
                      sg0, sg1, ss0, ss1):
        sid = lax.axis_index("s")
        wid = sid * NC + lax.axis_index("c")
        pltpu.sync_copy(idx_hbm.at[wid], idx_v)
        base = wid * b_per_w
        bufs = (rows0, rows1)
        gsems = (sg0, sg1)
        ssems = (ss0, ss1)

        for b_ in range(NBUF):
            pltpu.async_copy(ptab_hbm.at[idx_v.at[b_]], bufs[b_], gsems[b_])

        @pl.loop(0, n_chunks, step=NBUF)
        def _loop(j0):
            for b_ in range(NBUF):
                j = j0 + b_
                buf, sg, ss = bufs[b_], gsems[b_], ssems[b_]
                out_slc = out_hbm.at[pl.ds(base + j * CHUNK, CHUNK)]
                pltpu.make_async_copy(ptab_hbm.at[idx_v.at[j]], buf, sg).wait()
                pltpu.async_copy(buf, out_slc, ss)
                @pl.when(j + NBUF < n_chunks)
                def _():
                    pltpu.make_async_copy(buf, out_slc, ss).wait()
                    pltpu.async_copy(ptab_hbm.at[idx_v.at[j + NBUF]], buf, sg)

        for b_ in range(NBUF):
            j = n_chunks - NBUF + b_
            pltpu.make_async_copy(
                bufs[b_], out_hbm.at[pl.ds(base + j * CHUNK, CHUNK)],
                ssems[b_]).wait()

    return gather_kernel(ptab, idx.reshape(NW, n_chunks, CHUNK))


BB = 512


def _lookup_body(p_ref, src_ref, out_ref):
    ptab = p_ref[...]
    idx = src_ref[...]
    nb, n_s = idx.shape
    v = ptab.shape[0]
    iota = lax.broadcasted_iota(jnp.int32, (nb, v), 1)
    for s in range(n_s):
        col = idx[:, s:s + 1]
        onehot = (col == iota).astype(jnp.float32)
        out_ref[:, s, :] = jnp.dot(
            onehot, ptab, precision=lax.Precision.HIGHEST,
            preferred_element_type=jnp.float32)


def _tc_lookup(ptab, src, d_model):
    B, S = src.shape
    V = ptab.shape[0]
    return pl.pallas_call(
        _lookup_body,
        out_shape=jax.ShapeDtypeStruct((B, S, d_model), jnp.float32),
        grid=(B // BB,),
        in_specs=[
            pl.BlockSpec((V, d_model), lambda i: (0, 0)),
            pl.BlockSpec((BB, S), lambda i: (i, 0)),
        ],
        out_specs=pl.BlockSpec((BB, S, d_model), lambda i: (i, 0, 0)),
    )(ptab, src)


def kernel(src, table, W, b):
    B, S = src.shape
    D = W.shape[0]
    V = table.shape[0]
    v_pad = ((V + 2 * CHUNK - 1) // (2 * CHUNK)) * (2 * CHUNK)
    table_p = jnp.pad(table, ((0, v_pad - V), (0, 0)))
    ptab = _project_table(table_p, W, b)
    return _tc_lookup(ptab, src.astype(jnp.int32), D)


# speedup vs baseline: 1.4554x; 1.4554x over previous
import functools

import jax
import jax.numpy as jnp
from jax import lax
from jax.experimental import pallas as pl
from jax.experimental.pallas import tpu as pltpu
from jax.experimental.pallas import tpu_sc as plsc

NC = 2
NS = 16
NW = NC * NS

CHUNK = 64
NBUF = 2


def _proj_body(table_ref, w_ref, b_ref, out_ref):
    out_ref[...] = lax.dot_general(
        table_ref[...], w_ref[...],
        dimension_numbers=(((1,), (1,)), ((), ())),
        preferred_element_type=jnp.float32,
    ) + b_ref[...]


def _project_table(table, W, b):
    V, _ = table.shape
    D = W.shape[0]
    return pl.pallas_call(
        _proj_body,
        out_shape=jax.ShapeDtypeStruct((V, D), jnp.float32),
    )(table, W, b[None, :])


def _sc_gather(ptab, idx, n_total, d_model):
    b_per_w = n_total // NW
    n_chunks = b_per_w // CHUNK
    mesh = plsc.VectorSubcoreMesh(core_axis_name="c", subcore_axis_name="s")

    @functools.partial(
        pl.kernel,
        out_type=jax.ShapeDtypeStruct((n_total, d_model), jnp.float32),
        mesh=mesh,
        scratch_types=[
            pltpu.VMEM((n_chunks, CHUNK), jnp.int32),
            pltpu.VMEM((CHUNK, d_model), jnp.float32),
            pltpu.VMEM((CHUNK, d_model), jnp.float32),
            pltpu.SemaphoreType.DMA,
            pltpu.SemaphoreType.DMA,
            pltpu.SemaphoreType.DMA,
            pltpu.SemaphoreType.DMA,
        ],
    )
    def gather_kernel(ptab_hbm, idx_hbm, out_hbm, idx_v, rows0, rows1,
                      sg0, sg1, ss0, ss1):
        sid = lax.axis_index("s")
        wid = sid * NC + lax.axis_index("c")
        pltpu.sync_copy(idx_hbm.at[wid], idx_v)
        base = wid * b_per_w
        bufs = (rows0, rows1)
        gsems = (sg0, sg1)
        ssems = (ss0, ss1)

        for b_ in range(NBUF):
            pltpu.async_copy(ptab_hbm.at[idx_v.at[b_]], bufs[b_], gsems[b_])

        @pl.loop(0, n_chunks, step=NBUF)
        def _loop(j0):
            for b_ in range(NBUF):
                j = j0 + b_
                buf, sg, ss = bufs[b_], gsems[b_], ssems[b_]
                out_slc = out_hbm.at[pl.ds(base + j * CHUNK, CHUNK)]
                pltpu.make_async_copy(ptab_hbm.at[idx_v.at[j]], buf, sg).wait()
                pltpu.async_copy(buf, out_slc, ss)
                @pl.when(j + NBUF < n_chunks)
                def _():
                    pltpu.make_async_copy(buf, out_slc, ss).wait()
                    pltpu.async_copy(ptab_hbm.at[idx_v.at[j + NBUF]], buf, sg)

        for b_ in range(NBUF):
            j = n_chunks - NBUF + b_
            pltpu.make_async_copy(
                bufs[b_], out_hbm.at[pl.ds(base + j * CHUNK, CHUNK)],
                ssems[b_]).wait()

    return gather_kernel(ptab, idx.reshape(NW, n_chunks, CHUNK))


BB = 512


def _lookup_body(p_ref, src_ref, out_ref):
    ptab = p_ref[...]
    idx = src_ref[...]
    nb, n_s = idx.shape
    v = ptab.shape[0]
    ptab_hi = ptab.astype(jnp.bfloat16)
    ptab_lo = (ptab - ptab_hi.astype(jnp.float32)).astype(jnp.bfloat16)
    iota = lax.broadcasted_iota(jnp.int32, (nb, v), 1)
    for s in range(n_s):
        col = idx[:, s:s + 1]
        onehot = (col == iota).astype(jnp.bfloat16)
        out_ref[:, s, :] = jnp.dot(
            onehot, ptab_hi, preferred_element_type=jnp.float32) + jnp.dot(
            onehot, ptab_lo, preferred_element_type=jnp.float32)


def _tc_lookup(ptab, src, d_model):
    B, S = src.shape
    V = ptab.shape[0]
    return pl.pallas_call(
        _lookup_body,
        out_shape=jax.ShapeDtypeStruct((B, S, d_model), jnp.float32),
        grid=(B // BB,),
        in_specs=[
            pl.BlockSpec((V, d_model), lambda i: (0, 0)),
            pl.BlockSpec((BB, S), lambda i: (i, 0)),
        ],
        out_specs=pl.BlockSpec((BB, S, d_model), lambda i: (i, 0, 0)),
    )(ptab, src)


def kernel(src, table, W, b):
    B, S = src.shape
    D = W.shape[0]
    V = table.shape[0]
    v_pad = ((V + 2 * CHUNK - 1) // (2 * CHUNK)) * (2 * CHUNK)
    table_p = jnp.pad(table, ((0, v_pad - V), (0, 0)))
    ptab = _project_table(table_p, W, b)
    return _tc_lookup(ptab, src.astype(jnp.int32), D)


# speedup vs baseline: 1.5277x; 1.0497x over previous
import functools

import jax
import jax.numpy as jnp
from jax import lax
from jax.experimental import pallas as pl
from jax.experimental.pallas import tpu as pltpu
from jax.experimental.pallas import tpu_sc as plsc

NC = 2
NS = 16
NW = NC * NS

CHUNK = 64
NBUF = 2


def _proj_body(table_ref, w_ref, b_ref, out_ref):
    out_ref[...] = lax.dot_general(
        table_ref[...], w_ref[...],
        dimension_numbers=(((1,), (1,)), ((), ())),
        preferred_element_type=jnp.float32,
    ) + b_ref[...]


def _project_table(table, W, b):
    V, _ = table.shape
    D = W.shape[0]
    return pl.pallas_call(
        _proj_body,
        out_shape=jax.ShapeDtypeStruct((V, D), jnp.float32),
    )(table, W, b[None, :])


def _sc_gather(ptab, idx, n_total, d_model):
    b_per_w = n_total // NW
    n_chunks = b_per_w // CHUNK
    mesh = plsc.VectorSubcoreMesh(core_axis_name="c", subcore_axis_name="s")

    @functools.partial(
        pl.kernel,
        out_type=jax.ShapeDtypeStruct((n_total, d_model), jnp.float32),
        mesh=mesh,
        scratch_types=[
            pltpu.VMEM((n_chunks, CHUNK), jnp.int32),
            pltpu.VMEM((CHUNK, d_model), jnp.float32),
            pltpu.VMEM((CHUNK, d_model), jnp.float32),
            pltpu.SemaphoreType.DMA,
            pltpu.SemaphoreType.DMA,
            pltpu.SemaphoreType.DMA,
            pltpu.SemaphoreType.DMA,
        ],
    )
    def gather_kernel(ptab_hbm, idx_hbm, out_hbm, idx_v, rows0, rows1,
                      sg0, sg1, ss0, ss1):
        sid = lax.axis_index("s")
        wid = sid * NC + lax.axis_index("c")
        pltpu.sync_copy(idx_hbm.at[wid], idx_v)
        base = wid * b_per_w
        bufs = (rows0, rows1)
        gsems = (sg0, sg1)
        ssems = (ss0, ss1)

        for b_ in range(NBUF):
            pltpu.async_copy(ptab_hbm.at[idx_v.at[b_]], bufs[b_], gsems[b_])

        @pl.loop(0, n_chunks, step=NBUF)
        def _loop(j0):
            for b_ in range(NBUF):
                j = j0 + b_
                buf, sg, ss = bufs[b_], gsems[b_], ssems[b_]
                out_slc = out_hbm.at[pl.ds(base + j * CHUNK, CHUNK)]
                pltpu.make_async_copy(ptab_hbm.at[idx_v.at[j]], buf, sg).wait()
                pltpu.async_copy(buf, out_slc, ss)
                @pl.when(j + NBUF < n_chunks)
                def _():
                    pltpu.make_async_copy(buf, out_slc, ss).wait()
                    pltpu.async_copy(ptab_hbm.at[idx_v.at[j + NBUF]], buf, sg)

        for b_ in range(NBUF):
            j = n_chunks - NBUF + b_
            pltpu.make_async_copy(
                bufs[b_], out_hbm.at[pl.ds(base + j * CHUNK, CHUNK)],
                ssems[b_]).wait()

    return gather_kernel(ptab, idx.reshape(NW, n_chunks, CHUNK))


BB = 512


def _lookup_body(p_ref, src_ref, out_ref):
    ptab = p_ref[...]
    idx = src_ref[...]
    nb, n_s = idx.shape
    v = ptab.shape[0]
    iota = lax.broadcasted_iota(jnp.int32, (nb, v), 1)
    for s in range(n_s):
        col = idx[:, s:s + 1]
        onehot = (col == iota).astype(jnp.float32)
        out_ref[:, s, :] = jnp.dot(
            onehot, ptab, preferred_element_type=jnp.float32)


def _tc_lookup(ptab, src, d_model):
    B, S = src.shape
    V = ptab.shape[0]
    return pl.pallas_call(
        _lookup_body,
        out_shape=jax.ShapeDtypeStruct((B, S, d_model), jnp.float32),
        grid=(B // BB,),
        in_specs=[
            pl.BlockSpec((V, d_model), lambda i: (0, 0)),
            pl.BlockSpec((BB, S), lambda i: (i, 0)),
        ],
        out_specs=pl.BlockSpec((BB, S, d_model), lambda i: (i, 0, 0)),
    )(ptab, src)


def kernel(src, table, W, b):
    B, S = src.shape
    D = W.shape[0]
    V = table.shape[0]
    v_pad = ((V + 2 * CHUNK - 1) // (2 * CHUNK)) * (2 * CHUNK)
    table_p = jnp.pad(table, ((0, v_pad - V), (0, 0)))
    ptab = _project_table(table_p, W, b)
    return _tc_lookup(ptab, src.astype(jnp.int32), D)


# speedup vs baseline: 1.5295x; 1.0012x over previous
import functools

import jax
import jax.numpy as jnp
from jax import lax
from jax.experimental import pallas as pl
from jax.experimental.pallas import tpu as pltpu
from jax.experimental.pallas import tpu_sc as plsc

NC = 2
NS = 16
NW = NC * NS

CHUNK = 64
NBUF = 2


def _proj_body(table_ref, w_ref, b_ref, out_ref):
    out_ref[...] = lax.dot_general(
        table_ref[...], w_ref[...],
        dimension_numbers=(((1,), (1,)), ((), ())),
        preferred_element_type=jnp.float32,
    ) + b_ref[...]


def _project_table(table, W, b):
    V, _ = table.shape
    D = W.shape[0]
    return pl.pallas_call(
        _proj_body,
        out_shape=jax.ShapeDtypeStruct((V, D), jnp.float32),
    )(table, W, b[None, :])


def _sc_gather(ptab, idx, n_total, d_model):
    b_per_w = n_total // NW
    n_chunks = b_per_w // CHUNK
    mesh = plsc.VectorSubcoreMesh(core_axis_name="c", subcore_axis_name="s")

    @functools.partial(
        pl.kernel,
        out_type=jax.ShapeDtypeStruct((n_total, d_model), jnp.float32),
        mesh=mesh,
        scratch_types=[
            pltpu.VMEM((n_chunks, CHUNK), jnp.int32),
            pltpu.VMEM((CHUNK, d_model), jnp.float32),
            pltpu.VMEM((CHUNK, d_model), jnp.float32),
            pltpu.SemaphoreType.DMA,
            pltpu.SemaphoreType.DMA,
            pltpu.SemaphoreType.DMA,
            pltpu.SemaphoreType.DMA,
        ],
    )
    def gather_kernel(ptab_hbm, idx_hbm, out_hbm, idx_v, rows0, rows1,
                      sg0, sg1, ss0, ss1):
        sid = lax.axis_index("s")
        wid = sid * NC + lax.axis_index("c")
        pltpu.sync_copy(idx_hbm.at[wid], idx_v)
        base = wid * b_per_w
        bufs = (rows0, rows1)
        gsems = (sg0, sg1)
        ssems = (ss0, ss1)

        for b_ in range(NBUF):
            pltpu.async_copy(ptab_hbm.at[idx_v.at[b_]], bufs[b_], gsems[b_])

        @pl.loop(0, n_chunks, step=NBUF)
        def _loop(j0):
            for b_ in range(NBUF):
                j = j0 + b_
                buf, sg, ss = bufs[b_], gsems[b_], ssems[b_]
                out_slc = out_hbm.at[pl.ds(base + j * CHUNK, CHUNK)]
                pltpu.make_async_copy(ptab_hbm.at[idx_v.at[j]], buf, sg).wait()
                pltpu.async_copy(buf, out_slc, ss)
                @pl.when(j + NBUF < n_chunks)
                def _():
                    pltpu.make_async_copy(buf, out_slc, ss).wait()
                    pltpu.async_copy(ptab_hbm.at[idx_v.at[j + NBUF]], buf, sg)

        for b_ in range(NBUF):
            j = n_chunks - NBUF + b_
            pltpu.make_async_copy(
                bufs[b_], out_hbm.at[pl.ds(base + j * CHUNK, CHUNK)],
                ssems[b_]).wait()

    return gather_kernel(ptab, idx.reshape(NW, n_chunks, CHUNK))


BB = 512


def _lookup_body(p_ref, src_ref, out_ref):
    ptab = p_ref[...]
    idx = src_ref[...]
    nb, n_s = idx.shape
    v = ptab.shape[0]
    iota = lax.broadcasted_iota(jnp.int32, (nb, v), 1)
    for s in range(n_s):
        col = idx[:, s:s + 1]
        onehot = (col == iota).astype(jnp.float32)
        out_ref[:, s, :] = jnp.dot(
            onehot, ptab, preferred_element_type=jnp.float32)


def _tc_lookup(ptab, src, d_model):
    B, S = src.shape
    V = ptab.shape[0]
    return pl.pallas_call(
        _lookup_body,
        out_shape=jax.ShapeDtypeStruct((B, S, d_model), jnp.float32),
        grid=(B // BB,),
        in_specs=[
            pl.BlockSpec((V, d_model), lambda i: (0, 0)),
            pl.BlockSpec((BB, S), lambda i: (i, 0)),
        ],
        out_specs=pl.BlockSpec((BB, S, d_model), lambda i: (i, 0, 0)),
    )(ptab, src)


def kernel(src, table, W, b):
    D = W.shape[0]
    V = table.shape[0]
    v_pad = 128
    table_p = jnp.pad(table, ((0, v_pad - V), (0, 0)))
    ptab = _project_table(table_p, W, b)
    return _tc_lookup(ptab, src.astype(jnp.int32), D)
